# Initial kernel scaffold; baseline (speedup 1.0000x reference)
import jax
import jax.numpy as jnp
from jax.experimental import pallas as pl


def kernel(x, edge_index, W1l, b1l, W1u, b1u, W2l, b2l, W2u, b2u, W3l, b3l, W3u, b3u, Ws, bs, Wst, bst, Wo, bo):
    raise NotImplementedError("write your pallas kernel here")


# speedup vs baseline: 10.5264x; 10.5264x over previous
import jax
import jax.numpy as jnp
from jax import lax
from jax.experimental import pallas as pl
from jax.experimental.pallas import tpu as pltpu
from jax.experimental.pallas import tpu_sc as plsc

N = 50000
E = 800000
BLK_E = 128
K2 = 392
K1 = K2 // 2
EP = 16 * K2 * BLK_E
NACC = 50048
RPT = NACC // 16
ZR = 256
TC_BLK = 1000
TC_GRID = N // TC_BLK

_MESH = dict(core_axis_name="c", subcore_axis_name="s")


def _make_segsum_body(K, CHB):
    nch = K // CHB
    assert nch % 2 == 0 and CHB % 2 == 0

    def body(xref, srcref, dstref, zref, outref, sbuf, dbuf, rbuf, acc,
             g0, g1, is0, is1, id0, id1):
        c = lax.axis_index("c")
        s = lax.axis_index("s")
        zbase = s * RPT
        nfull = RPT // ZR
        rem = RPT - nfull * ZR
        slot0 = rbuf.at[pl.ds(0, BLK_E)]
        slot1 = rbuf.at[pl.ds(BLK_E, BLK_E)]

        def ichunk(k, half, ssem, dsem):
            sl = pl.ds(half * CHB, CHB)
            hb = pl.ds(k * CHB, CHB)
            pltpu.async_copy(srcref.at[c, s, hb], sbuf.at[sl], ssem)
            pltpu.async_copy(dstref.at[c, s, hb], dbuf.at[sl], dsem)

        def iwait(ssem, dsem):
            pltpu.make_async_copy(srcref.at[c, s, pl.ds(0, CHB)],
                                  sbuf.at[pl.ds(0, CHB)], ssem).wait()
            pltpu.make_async_copy(dstref.at[c, s, pl.ds(0, CHB)],
                                  dbuf.at[pl.ds(0, CHB)], dsem).wait()

        ichunk(0, 0, is0, id0)

        pltpu.sync_copy(zref, rbuf)

        def zstep(k, _):
            pltpu.sync_copy(rbuf, acc.at[pl.ds(zbase + k * ZR, ZR)])
            return _

        lax.fori_loop(0, nfull, zstep, None)
        pltpu.sync_copy(rbuf.at[pl.ds(0, rem)],
                        acc.at[pl.ds(zbase + nfull * ZR, rem)])
        plsc.subcore_barrier()

        def gstart(j, slot, sem):
            pltpu.async_copy(xref.at[sbuf.at[j]], slot, sem)

        def gwait(j, slot, sem):
            pltpu.make_async_copy(xref.at[sbuf.at[j]], slot, sem).wait()

        def process(half):
            off = half * CHB
            gstart(off, slot0, g0)

            def estep(i, _):
                j0 = off + 2 * i
                j1 = j0 + 1
                gstart(j1, slot1, g1)
                gwait(j0, slot0, g0)
                pltpu.sync_copy(slot0, acc.at[dbuf.at[j0]], add=True)

                @pl.when(i < CHB // 2 - 1)
                def _():
                    gstart(j0 + 2, slot0, g0)

                gwait(j1, slot1, g1)
                pltpu.sync_copy(slot1, acc.at[dbuf.at[j1]], add=True)
                return _

            lax.fori_loop(0, CHB // 2, estep, None)

        def cpair(t, _):
            ichunk(2 * t + 1, 1, is1, id1)
            iwait(is0, id0)
            process(0)

            @pl.when(t < nch // 2 - 1)
            def _():
                ichunk(2 * t + 2, 0, is0, id0)

            iwait(is1, id1)
            process(1)
            return _

        lax.fori_loop(0, nch // 2, cpair, None)
        plsc.subcore_barrier()

        @pl.when(s < 15)
        def _():
            pltpu.sync_copy(acc.at[pl.ds(zbase, RPT)],
                            outref.at[c, pl.ds(zbase, RPT)])

        @pl.when(s == 15)
        def _():
            last = N - 15 * RPT
            pltpu.sync_copy(acc.at[pl.ds(zbase, last)],
                            outref.at[c, pl.ds(zbase, last)])

    return body


def _segsum(xflat, srcq, dstq, zrows, width, K, CHB):
    return pl.kernel(
        _make_segsum_body(K, CHB),
        out_type=jax.ShapeDtypeStruct((2, N, width), jnp.float32),
        mesh=plsc.VectorSubcoreMesh(**_MESH),
        scratch_types=[
            pltpu.VMEM((2 * CHB, BLK_E), jnp.int32),
            pltpu.VMEM((2 * CHB, BLK_E), jnp.int32),
            pltpu.VMEM((ZR, width), jnp.float32),
            pltpu.VMEM_SHARED((NACC, width), jnp.float32),
            pltpu.SemaphoreType.DMA,
            pltpu.SemaphoreType.DMA,
            pltpu.SemaphoreType.DMA,
            pltpu.SemaphoreType.DMA,
            pltpu.SemaphoreType.DMA,
            pltpu.SemaphoreType.DMA,
        ],
        compiler_params=pltpu.CompilerParams(use_tc_tiling_on_sc=False),
    )(xflat, srcq, dstq, zrows)


def _dot(a, b):
    return jnp.dot(a, b, preferred_element_type=jnp.float32,
                   precision=jax.lax.Precision.HIGHEST)


def _layer1_body(x_ref, s_ref, wl_ref, bl_ref, wt_ref, wb_ref, bu_ref,
                 h_ref, cnt_ref):
    xb = x_ref[...]
    st = s_ref[0] + s_ref[1] + xb
    cnt = st[:, 5:6]
    m = st / cnt
    aggr = _dot(m, wl_ref[...]) + bl_ref[...]
    h = jnp.maximum(_dot(xb, wt_ref[...]) + _dot(aggr, wb_ref[...]) + bu_ref[...], 0.0)
    h_ref[0] = h[:, :32]
    h_ref[1] = h[:, 32:]
    cnt_ref[...] = cnt


def _layer2_body(h_ref, s_ref, cnt_ref, wl_ref, bl_ref, wt_ref, wb_ref, bu_ref,
                 out_ref):
    h = jnp.concatenate([h_ref[0], h_ref[1]], axis=1)
    st = jnp.concatenate([s_ref[0], s_ref[1]], axis=1) + h
    m = st / cnt_ref[...]
    aggr = _dot(m, wl_ref[...]) + bl_ref[...]
    hn = jnp.maximum(_dot(h, wt_ref[...]) + _dot(aggr, wb_ref[...]) + bu_ref[...], 0.0)
    out_ref[0] = hn[:, :32]
    out_ref[1] = hn[:, 32:]


def _layer3_body(h_ref, s_ref, cnt_ref, wl_ref, bl_ref, wt_ref, wb_ref, bu_ref,
                 ws_ref, bs_ref, wst_ref, bst_ref, wo_ref, bo_ref,
                 hout_ref, state_ref, stab_ref, opf_ref):
    i = pl.program_id(0)
    h = jnp.concatenate([h_ref[0], h_ref[1]], axis=1)
    st = jnp.concatenate([s_ref[0], s_ref[1]], axis=1) + h
    m = st / cnt_ref[...]
    aggr = _dot(m, wl_ref[...]) + bl_ref[...]
    hn = jnp.maximum(_dot(h, wt_ref[...]) + _dot(aggr, wb_ref[...]) + bu_ref[...], 0.0)
    hout_ref[...] = hn
    state_ref[...] = _dot(hn, ws_ref[...]) + bs_ref[...]
    z = _dot(hn, wst_ref[...]) + bst_ref[...]
    psig = jnp.sum(1.0 / (1.0 + jnp.exp(-z)))
    popf = jnp.sum(_dot(hn, wo_ref[...]) + bo_ref[...])

    @pl.when(i == 0)
    def _():
        stab_ref[...] = jnp.zeros_like(stab_ref)
        opf_ref[...] = jnp.zeros_like(opf_ref)

    stab_ref[...] += jnp.reshape(psig, (1, 1))
    opf_ref[...] += jnp.reshape(popf, (1, 1))

    @pl.when(i == TC_GRID - 1)
    def _():
        stab_ref[...] = stab_ref[...] * (1.0 / N)
        opf_ref[...] = opf_ref[...] * (1.0 / N)


def _wspec(r, c):
    return pl.BlockSpec((r, c), lambda i: (0, 0))


def kernel(x, edge_index, W1l, b1l, W1u, b1u, W2l, b2l, W2u, b2u,
           W3l, b3l, W3u, b3u, Ws, bs, Wst, bst, Wo, bo):
    f32 = jnp.float32
    x8 = jnp.concatenate([x, jnp.ones((N, 1), f32), jnp.zeros((N, 2), f32)], axis=1)

    pad = EP - E
    srcp = jnp.concatenate([edge_index[0], jnp.zeros((pad,), jnp.int32)])
    dstp = jnp.concatenate([edge_index[1], jnp.full((pad,), N, jnp.int32)])
    srcA = srcp.reshape(2, 16, K1, BLK_E)
    dstA = dstp.reshape(2, 16, K1, BLK_E)
    srcB = (srcp[None, :] + jnp.array([[0], [N]], jnp.int32)).reshape(2, 16, K2, BLK_E)
    dstB = jnp.broadcast_to(dstp.reshape(1, 16, K2, BLK_E), (2, 16, K2, BLK_E))
    z8 = jnp.zeros((ZR, 8), f32)
    z32 = jnp.zeros((ZR, 32), f32)

    w1l_p = jnp.concatenate([W1l, jnp.zeros((3, 64), f32)], axis=0)
    w1u_t = jnp.concatenate([W1u[:5], jnp.zeros((3, 64), f32)], axis=0)
    w1u_b = W1u[5:]
    r1 = lambda v: v.reshape(1, -1)

    s1 = _segsum(x8, srcA, dstA, z8, 8, K1, 98)
    h1s, cnt = pl.pallas_call(
        _layer1_body,
        grid=(TC_GRID,),
        in_specs=[
            pl.BlockSpec((TC_BLK, 8), lambda i: (i, 0)),
            pl.BlockSpec((2, TC_BLK, 8), lambda i: (0, i, 0)),
            _wspec(8, 64), _wspec(1, 64), _wspec(8, 64), _wspec(64, 64), _wspec(1, 64),
        ],
        out_specs=[
            pl.BlockSpec((2, TC_BLK, 32), lambda i: (0, i, 0)),
            pl.BlockSpec((TC_BLK, 1), lambda i: (i, 0)),
        ],
        out_shape=[
            jax.ShapeDtypeStruct((2, N, 32), f32),
            jax.ShapeDtypeStruct((N, 1), f32),
        ],
    )(x8, s1, w1l_p, r1(b1l), w1u_t, w1u_b, r1(b1u))

    s2 = _segsum(h1s.reshape(2 * N, 32), srcB, dstB, z32, 32, K2, 28)
    h2s = pl.pallas_call(
        _layer2_body,
        grid=(TC_GRID,),
        in_specs=[
            pl.BlockSpec((2, TC_BLK, 32), lambda i: (0, i, 0)),
            pl.BlockSpec((2, TC_BLK, 32), lambda i: (0, i, 0)),
            pl.BlockSpec((TC_BLK, 1), lambda i: (i, 0)),
            _wspec(64, 64), _wspec(1, 64), _wspec(64, 64), _wspec(64, 64), _wspec(1, 64),
        ],
        out_specs=pl.BlockSpec((2, TC_BLK, 32), lambda i: (0, i, 0)),
        out_shape=jax.ShapeDtypeStruct((2, N, 32), f32),
    )(h1s, s2, cnt, W2l, r1(b2l), W2u[:64], W2u[64:], r1(b2u))

    s3 = _segsum(h2s.reshape(2 * N, 32), srcB, dstB, z32, 32, K2, 28)
    h, state, stab, opf = pl.pallas_call(
        _layer3_body,
        grid=(TC_GRID,),
        in_specs=[
            pl.BlockSpec((2, TC_BLK, 32), lambda i: (0, i, 0)),
            pl.BlockSpec((2, TC_BLK, 32), lambda i: (0, i, 0)),
            pl.BlockSpec((TC_BLK, 1), lambda i: (i, 0)),
            _wspec(64, 64), _wspec(1, 64), _wspec(64, 64), _wspec(64, 64), _wspec(1, 64),
            _wspec(64, 2), _wspec(1, 2), _wspec(64, 1), _wspec(1, 1),
            _wspec(64, 1), _wspec(1, 1),
        ],
        out_specs=[
            pl.BlockSpec((TC_BLK, 64), lambda i: (i, 0)),
            pl.BlockSpec((TC_BLK, 2), lambda i: (i, 0)),
            pl.BlockSpec((1, 1), lambda i: (0, 0)),
            pl.BlockSpec((1, 1), lambda i: (0, 0)),
        ],
        out_shape=[
            jax.ShapeDtypeStruct((N, 64), f32),
            jax.ShapeDtypeStruct((N, 2), f32),
            jax.ShapeDtypeStruct((1, 1), f32),
            jax.ShapeDtypeStruct((1, 1), f32),
        ],
        compiler_params=pltpu.CompilerParams(
            dimension_semantics=("arbitrary",)),
    )(h2s, s3, cnt, W3l, r1(b3l), W3u[:64], W3u[64:], r1(b3u),
      Ws, r1(bs), Wst, r1(bst), Wo, r1(bo))

    return (state, stab[0, 0], opf[0, 0], h)


# speedup vs baseline: 12.6032x; 1.1973x over previous
import jax
import jax.numpy as jnp
from jax import lax
from jax.experimental import pallas as pl
from jax.experimental.pallas import tpu as pltpu
from jax.experimental.pallas import tpu_sc as plsc

N = 50000
E = 800000
BLK_E = 128
K2 = 400
K1 = K2 // 2
CHB = 10
RING = 5
EP = 16 * K2 * BLK_E
NACC = 50048
RPT = NACC // 16
ZR = RING * BLK_E
TC_BLK = 2000
TC_GRID = N // TC_BLK

_MESH = dict(core_axis_name="c", subcore_axis_name="s")


def _make_segsum_body(K):
    nch = K // CHB
    assert nch % 2 == 0 and K == nch * CHB

    def body(xref, srcref, dstref, zref, outref, sbuf, dbuf, rbuf, acc, *sems):
        gs = sems[0:RING]
        ss = sems[RING:2 * RING]
        is0, is1, id0, id1 = sems[2 * RING:]
        c = lax.axis_index("c")
        s = lax.axis_index("s")
        zbase = s * RPT
        nfull = RPT // ZR
        rem = RPT - nfull * ZR
        dst4 = len(dstref.shape) == 4

        def ichunk(k, half, ssem, dsem):
            sl = pl.ds(half * CHB, CHB)
            hb = pl.ds(k * CHB, CHB)
            pltpu.async_copy(srcref.at[c, s, hb], sbuf.at[sl], ssem)
            dref = dstref.at[c, s, hb] if dst4 else dstref.at[s, hb]
            pltpu.async_copy(dref, dbuf.at[sl], dsem)

        def iwait(ssem, dsem):
            hb = pl.ds(0, CHB)
            pltpu.make_async_copy(srcref.at[c, s, hb],
                                  sbuf.at[pl.ds(0, CHB)], ssem).wait()
            dref = dstref.at[c, s, hb] if dst4 else dstref.at[s, hb]
            pltpu.make_async_copy(dref, dbuf.at[pl.ds(0, CHB)], dsem).wait()

        ichunk(0, 0, is0, id0)

        pltpu.sync_copy(zref, rbuf)

        def zstep(k, _):
            pltpu.sync_copy(rbuf, acc.at[pl.ds(zbase + k * ZR, ZR)])
            return _

        lax.fori_loop(0, nfull, zstep, None)
        pltpu.sync_copy(rbuf.at[pl.ds(0, rem)],
                        acc.at[pl.ds(zbase + nfull * ZR, rem)])
        plsc.subcore_barrier()

        def process(half):
            off = half * CHB

            def slot(q):
                return rbuf.at[pl.ds(q * BLK_E, BLK_E)]

            def gpair(j):
                q = (j % RING)
                return (xref.at[sbuf.at[off + j]], slot(q), gs[q])

            def spair(j):
                q = (j % RING)
                return (slot(q), acc.at[dbuf.at[off + j]], ss[q])

            for j in range(3):
                pltpu.async_copy(*gpair(j))
            for j in range(CHB):
                src, dst, sem = gpair(j)
                pltpu.make_async_copy(src, dst, sem).wait()
                ssrc, sdst, ssem = spair(j)
                pltpu.async_copy(ssrc, sdst, ssem, add=True)
                if j + 3 < CHB:
                    if j >= 2:
                        wsrc, wdst, wsem = spair(j - 2)
                        pltpu.make_async_copy(wsrc, wdst, wsem).wait()
                    pltpu.async_copy(*gpair(j + 3))
            for j in range(max(CHB - 5, 0), CHB):
                wsrc, wdst, wsem = spair(j)
                pltpu.make_async_copy(wsrc, wdst, wsem).wait()

        def cpair(t, _):
            ichunk(2 * t + 1, 1, is1, id1)
            iwait(is0, id0)
            process(0)

            @pl.when(t < nch // 2 - 1)
            def _():
                ichunk(2 * t + 2, 0, is0, id0)

            iwait(is1, id1)
            process(1)
            return _

        lax.fori_loop(0, nch // 2, cpair, None)
        plsc.subcore_barrier()

        @pl.when(s < 15)
        def _():
            pltpu.sync_copy(acc.at[pl.ds(zbase, RPT)],
                            outref.at[c, pl.ds(zbase, RPT)])

        @pl.when(s == 15)
        def _():
            last = N - 15 * RPT
            pltpu.sync_copy(acc.at[pl.ds(zbase, last)],
                            outref.at[c, pl.ds(zbase, last)])

    return body


def _segsum(xflat, srcq, dstq, zrows, width, K):
    return pl.kernel(
        _make_segsum_body(K),
        out_type=jax.ShapeDtypeStruct((2, N, width), jnp.float32),
        mesh=plsc.VectorSubcoreMesh(**_MESH),
        scratch_types=(
            [
                pltpu.VMEM((2 * CHB, BLK_E), jnp.int32),
                pltpu.VMEM((2 * CHB, BLK_E), jnp.int32),
                pltpu.VMEM((ZR, width), jnp.float32),
                pltpu.VMEM_SHARED((NACC, width), jnp.float32),
            ]
            + [pltpu.SemaphoreType.DMA] * (2 * RING + 4)
        ),
        compiler_params=pltpu.CompilerParams(use_tc_tiling_on_sc=False),
    )(xflat, srcq, dstq, zrows)


def _dot(a, b):
    return jnp.dot(a, b, preferred_element_type=jnp.float32)


def _layer1_body(x_ref, s_ref, wl_ref, bl_ref, wt_ref, wb_ref, bu_ref,
                 h_ref, cnt_ref):
    xb = x_ref[...]
    st = s_ref[0] + s_ref[1] + xb
    cnt = st[:, 5:6]
    m = st / cnt
    aggr = _dot(m, wl_ref[...]) + bl_ref[...]
    h = jnp.maximum(_dot(xb, wt_ref[...]) + _dot(aggr, wb_ref[...]) + bu_ref[...], 0.0)
    h_ref[0] = h[:, :32]
    h_ref[1] = h[:, 32:]
    cnt_ref[...] = cnt


def _layer2_body(h_ref, s_ref, cnt_ref, wl_ref, bl_ref, wt_ref, wb_ref, bu_ref,
                 out_ref):
    h = jnp.concatenate([h_ref[0], h_ref[1]], axis=1)
    st = jnp.concatenate([s_ref[0], s_ref[1]], axis=1) + h
    m = st / cnt_ref[...]
    aggr = _dot(m, wl_ref[...]) + bl_ref[...]
    hn = jnp.maximum(_dot(h, wt_ref[...]) + _dot(aggr, wb_ref[...]) + bu_ref[...], 0.0)
    out_ref[0] = hn[:, :32]
    out_ref[1] = hn[:, 32:]


def _layer3_body(h_ref, s_ref, cnt_ref, wl_ref, bl_ref, wt_ref, wb_ref, bu_ref,
                 ws_ref, bs_ref, wst_ref, bst_ref, wo_ref, bo_ref,
                 hout_ref, state_ref, stab_ref, opf_ref):
    i = pl.program_id(0)
    h = jnp.concatenate([h_ref[0], h_ref[1]], axis=1)
    st = jnp.concatenate([s_ref[0], s_ref[1]], axis=1) + h
    m = st / cnt_ref[...]
    aggr = _dot(m, wl_ref[...]) + bl_ref[...]
    hn = jnp.maximum(_dot(h, wt_ref[...]) + _dot(aggr, wb_ref[...]) + bu_ref[...], 0.0)
    hout_ref[...] = hn
    state_ref[...] = _dot(hn, ws_ref[...]) + bs_ref[...]
    z = _dot(hn, wst_ref[...]) + bst_ref[...]
    psig = jnp.sum(1.0 / (1.0 + jnp.exp(-z)))
    popf = jnp.sum(_dot(hn, wo_ref[...]) + bo_ref[...])

    @pl.when(i == 0)
    def _():
        stab_ref[...] = jnp.zeros_like(stab_ref)
        opf_ref[...] = jnp.zeros_like(opf_ref)

    stab_ref[...] += jnp.reshape(psig, (1, 1))
    opf_ref[...] += jnp.reshape(popf, (1, 1))

    @pl.when(i == TC_GRID - 1)
    def _():
        stab_ref[...] = stab_ref[...] * (1.0 / N)
        opf_ref[...] = opf_ref[...] * (1.0 / N)


def _wspec(r, c):
    return pl.BlockSpec((r, c), lambda i: (0, 0))


def kernel(x, edge_index, W1l, b1l, W1u, b1u, W2l, b2l, W2u, b2u,
           W3l, b3l, W3u, b3u, Ws, bs, Wst, bst, Wo, bo):
    f32 = jnp.float32
    x8 = jnp.concatenate([x, jnp.ones((N, 1), f32), jnp.zeros((N, 2), f32)], axis=1)

    pad = EP - E
    srcp = jnp.concatenate([edge_index[0], jnp.zeros((pad,), jnp.int32)])
    dstp = jnp.concatenate([edge_index[1], jnp.full((pad,), N, jnp.int32)])
    srcA = srcp.reshape(2, 16, K1, BLK_E)
    dstA = dstp.reshape(2, 16, K1, BLK_E)
    srcB = (srcp[None, :] + jnp.array([[0], [N]], jnp.int32)).reshape(2, 16, K2, BLK_E)
    dstB = dstp.reshape(16, K2, BLK_E)
    z8 = jnp.zeros((ZR, 8), f32)
    z32 = jnp.zeros((ZR, 32), f32)

    w1l_p = jnp.concatenate([W1l, jnp.zeros((3, 64), f32)], axis=0)
    w1u_t = jnp.concatenate([W1u[:5], jnp.zeros((3, 64), f32)], axis=0)
    w1u_b = W1u[5:]
    r1 = lambda v: v.reshape(1, -1)

    s1 = _segsum(x8, srcA, dstA, z8, 8, K1)
    h1s, cnt = pl.pallas_call(
        _layer1_body,
        grid=(TC_GRID,),
        in_specs=[
            pl.BlockSpec((TC_BLK, 8), lambda i: (i, 0)),
            pl.BlockSpec((2, TC_BLK, 8), lambda i: (0, i, 0)),
            _wspec(8, 64), _wspec(1, 64), _wspec(8, 64), _wspec(64, 64), _wspec(1, 64),
        ],
        out_specs=[
            pl.BlockSpec((2, TC_BLK, 32), lambda i: (0, i, 0)),
            pl.BlockSpec((TC_BLK, 1), lambda i: (i, 0)),
        ],
        out_shape=[
            jax.ShapeDtypeStruct((2, N, 32), f32),
            jax.ShapeDtypeStruct((N, 1), f32),
        ],
    )(x8, s1, w1l_p, r1(b1l), w1u_t, w1u_b, r1(b1u))

    s2 = _segsum(h1s.reshape(2 * N, 32), srcB, dstB, z32, 32, K2)
    h2s = pl.pallas_call(
        _layer2_body,
        grid=(TC_GRID,),
        in_specs=[
            pl.BlockSpec((2, TC_BLK, 32), lambda i: (0, i, 0)),
            pl.BlockSpec((2, TC_BLK, 32), lambda i: (0, i, 0)),
            pl.BlockSpec((TC_BLK, 1), lambda i: (i, 0)),
            _wspec(64, 64), _wspec(1, 64), _wspec(64, 64), _wspec(64, 64), _wspec(1, 64),
        ],
        out_specs=pl.BlockSpec((2, TC_BLK, 32), lambda i: (0, i, 0)),
        out_shape=jax.ShapeDtypeStruct((2, N, 32), f32),
    )(h1s, s2, cnt, W2l, r1(b2l), W2u[:64], W2u[64:], r1(b2u))

    s3 = _segsum(h2s.reshape(2 * N, 32), srcB, dstB, z32, 32, K2)
    h, state, stab, opf = pl.pallas_call(
        _layer3_body,
        grid=(TC_GRID,),
        in_specs=[
            pl.BlockSpec((2, TC_BLK, 32), lambda i: (0, i, 0)),
            pl.BlockSpec((2, TC_BLK, 32), lambda i: (0, i, 0)),
            pl.BlockSpec((TC_BLK, 1), lambda i: (i, 0)),
            _wspec(64, 64), _wspec(1, 64), _wspec(64, 64), _wspec(64, 64), _wspec(1, 64),
            _wspec(64, 2), _wspec(1, 2), _wspec(64, 1), _wspec(1, 1),
            _wspec(64, 1), _wspec(1, 1),
        ],
        out_specs=[
            pl.BlockSpec((TC_BLK, 64), lambda i: (i, 0)),
            pl.BlockSpec((TC_BLK, 2), lambda i: (i, 0)),
            pl.BlockSpec((1, 1), lambda i: (0, 0)),
            pl.BlockSpec((1, 1), lambda i: (0, 0)),
        ],
        out_shape=[
            jax.ShapeDtypeStruct((N, 64), f32),
            jax.ShapeDtypeStruct((N, 2), f32),
            jax.ShapeDtypeStruct((1, 1), f32),
            jax.ShapeDtypeStruct((1, 1), f32),
        ],
        compiler_params=pltpu.CompilerParams(
            dimension_semantics=("arbitrary",)),
    )(h2s, s3, cnt, W3l, r1(b3l), W3u[:64], W3u[64:], r1(b3u),
      Ws, r1(bs), Wst, r1(bst), Wo, r1(bo))

    return (state, stab[0, 0], opf[0, 0], h)
